# baseline (speedup 1.0000x reference)
import jax
import jax.numpy as jnp
from jax import lax
from jax.experimental import pallas as pl
from jax.experimental.pallas import tpu as pltpu


def _gather_block_kernel(idx_ref, off_ref, tab_ref, out_ref, *, fields, pairs_per_field):
    bsub = idx_ref.shape[0]
    d = out_ref.shape[2]
    g = idx_ref[...] + off_ref[...]
    for half in range(fields // 8):
        res = []
        for fi in range(8):
            f = half * 8 + fi
            base = f * pairs_per_field
            col = g[:, f : f + 1]
            pair_id = lax.shift_right_logical(col, 1)
            pair_ids = base + lax.broadcasted_iota(
                jnp.int32, (bsub, pairs_per_field), 1
            )
            onehot = (pair_id == pair_ids).astype(jnp.bfloat16)
            sub = tab_ref[base : base + pairs_per_field, :]
            pair = jnp.dot(onehot, sub, preferred_element_type=jnp.float32)
            odd = (col & 1) == 1
            res.append(jnp.where(odd, pair[:, d:], pair[:, :d]))
        stacked = jnp.stack(res, axis=0)
        out_ref[:, half * 8 : (half + 1) * 8, :] = jnp.transpose(stacked, (1, 0, 2))


def kernel(x, embedding_weight, offsets):
    B, F = x.shape
    V, D = embedding_weight.shape
    rows_per_field = V // F

    packed = embedding_weight.astype(jnp.bfloat16).reshape(V // 2, 2 * D)
    off_row = offsets.astype(jnp.int32).reshape(1, F)

    BSUB = 512
    assert B % BSUB == 0

    out = pl.pallas_call(
        lambda i, of, t, o: _gather_block_kernel(
            i, of, t, o, fields=F, pairs_per_field=rows_per_field // 2
        ),
        out_shape=jax.ShapeDtypeStruct((B, F, D), jnp.float32),
        grid=(B // BSUB,),
        in_specs=[
            pl.BlockSpec((BSUB, F), lambda i: (i, 0)),
            pl.BlockSpec((1, F), lambda i: (0, 0)),
            pl.BlockSpec((V // 2, 2 * D), lambda i: (0, 0)),
        ],
        out_specs=pl.BlockSpec((BSUB, F, D), lambda i: (i, 0, 0)),
        compiler_params=pltpu.CompilerParams(
            dimension_semantics=("parallel",),
            vmem_limit_bytes=48 * 1024 * 1024,
        ),
    )(x.astype(jnp.int32), off_row, packed)

    return out
